# baseline (speedup 1.0000x reference)
import functools

import jax
import jax.numpy as jnp
import numpy as np
from jax import lax
from jax.experimental import pallas as pl
from jax.experimental.pallas import tpu as pltpu
from jax.experimental.pallas import tpu_sc as plsc

B = 4
N = 8192
S = 1024
K = 32
CPAD = 32
C2 = 64
R2 = np.float32(0.04)
EPS = 1e-5
NC, NS = 2, 16
NW = NC * NS
RPW = (B * S) // NW
WPB = NW // B
GC = 16
NCHUNK = N // 16
RTOT = B * S * K
UNROLL = 4
SCANP = 128
RB = 2048
GB = RB // K


def _bf16_round(v):
    u = plsc.bitcast(v, jnp.uint32)
    lsb = lax.shift_right_logical(u, jnp.uint32(16)) & jnp.uint32(1)
    r = (u + jnp.uint32(0x7FFF) + lsb) & jnp.uint32(0xFFFF0000)
    return plsc.bitcast(r, jnp.float32)


def _sc_body(xyz_hbm, feat_hbm, out_hbm, xyz_v, pb_v, b2_v, idx_v, gbuf_v, sem):
    wid = lax.axis_index("s") * NC + lax.axis_index("c")
    b = wid // WPB
    s_base = (wid % WPB) * RPW
    pltpu.sync_copy(xyz_hbm.at[b], xyz_v)

    lane = jnp.arange(16, dtype=jnp.int32)

    def pre_body(ch, carry):
        n0 = ch * 16
        px = xyz_v[pl.ds(n0, 16)]
        py = xyz_v[pl.ds(N + n0, 16)]
        pz = xyz_v[pl.ds(2 * N + n0, 16)]
        pb_v[pl.ds(n0, 16)] = _bf16_round(px)
        pb_v[pl.ds(N + n0, 16)] = _bf16_round(py)
        pb_v[pl.ds(2 * N + n0, 16)] = _bf16_round(pz)
        b2_v[pl.ds(n0, 16)] = (px * px + py * py) + pz * pz
        return carry

    lax.fori_loop(0, NCHUNK, pre_body, 0)

    def centroid_body(ci, carry):
        s = s_base + ci
        sv = jnp.full((16,), s, jnp.int32)
        cx = plsc.load_gather(xyz_v, [sv])
        cy = plsc.load_gather(xyz_v, [sv + N])
        cz = plsc.load_gather(xyz_v, [sv + 2 * N])
        a2 = (cx * cx + cy * cy) + cz * cz
        cbx = _bf16_round(cx)
        cby = _bf16_round(cy)
        cbz = _bf16_round(cz)
        rowbase = ci * K

        def cond(c):
            chunk, cntv = c
            return jnp.logical_and(cntv[0] < K, chunk < NCHUNK // UNROLL)

        def body(c):
            chunk, cntv = c
            cv = cntv
            for u in range(UNROLL):
                n0 = chunk * (16 * UNROLL) + u * 16
                mm = (cbx * pb_v[pl.ds(n0, 16)] + cby * pb_v[pl.ds(N + n0, 16)]
                      ) + cbz * pb_v[pl.ds(2 * N + n0, 16)]
                d2 = ((-2.0 * mm) + a2) + b2_v[pl.ds(n0, 16)]
                m = d2 <= R2
                csum = plsc.cumsum(m.astype(jnp.int32))
                pos = cv + csum - 1
                ok = jnp.logical_and(m, pos < K)
                plsc.store_scatter(idx_v, [rowbase + pos], b * N + n0 + lane,
                                   mask=ok)
                cv = cv + plsc.all_reduce_population_count(m)
            return chunk + 1, cv

        _, cntv = lax.while_loop(
            cond, body, (jnp.int32(0), jnp.zeros((16,), jnp.int32)))
        cnt = cntv[0]
        first = plsc.load_gather(idx_v, [jnp.full((16,), rowbase, jnp.int32)])
        for h in range(2):
            lv = lane + (h * 16)
            plsc.store_scatter(idx_v, [rowbase + lv], first, mask=lv >= cnt)
        return carry

    lax.fori_loop(0, RPW, centroid_body, 0)

    out_base = (b * S + s_base) * K
    nrows = GC * K

    def gather_body(gi, carry):
        idx_sub = idx_v.at[pl.ds(gi * nrows, nrows)]
        pltpu.async_copy(feat_hbm.at[idx_sub], gbuf_v, sem).wait()
        pltpu.sync_copy(gbuf_v, out_hbm.at[pl.ds(out_base + gi * nrows, nrows)])
        return carry

    lax.fori_loop(0, RPW // GC, gather_body, 0)


def _sc_ball_gather(xyz, feat):
    mesh = plsc.VectorSubcoreMesh(
        core_axis_name="c", subcore_axis_name="s", num_cores=NC, num_subcores=NS)
    return pl.kernel(
        _sc_body,
        out_type=jax.ShapeDtypeStruct((RTOT, CPAD), jnp.float32),
        mesh=mesh,
        scratch_types=[
            pltpu.VMEM((3 * N,), jnp.float32),
            pltpu.VMEM((3 * N,), jnp.float32),
            pltpu.VMEM((N,), jnp.float32),
            pltpu.VMEM((RPW * K,), jnp.int32),
            pltpu.VMEM((GC * K, CPAD), jnp.float32),
            pltpu.SemaphoreType.DMA,
        ],
        compiler_params=pltpu.CompilerParams(
            needs_layout_passes=False, use_tc_tiling_on_sc=False),
    )(xyz.reshape(B, 3 * N), feat)


def _p1_body(x_ref, cent_ref, w_ref, y_ref, st_ref):
    i = pl.program_id(0)
    x = x_ref[...]
    xc = x.reshape(GB, K, CPAD) - cent_ref[...][:, None, :]
    y = jnp.dot(xc.reshape(RB, CPAD), w_ref[...],
                preferred_element_type=jnp.float32)
    y_ref[...] = y

    @pl.when(i == 0)
    def _():
        st_ref[...] = jnp.zeros_like(st_ref)

    st_ref[0:1, :] += jnp.sum(y, axis=0, keepdims=True)
    st_ref[1:2, :] += jnp.sum(y * y, axis=0, keepdims=True)


def _mid_body(y_ref, a_ref, c_ref, w_ref, o_ref, st_ref):
    i = pl.program_id(0)
    h = jnp.maximum(y_ref[...] * a_ref[...] + c_ref[...], 0.0)
    y = jnp.dot(h, w_ref[...], preferred_element_type=jnp.float32)
    o_ref[...] = y

    @pl.when(i == 0)
    def _():
        st_ref[...] = jnp.zeros_like(st_ref)

    st_ref[0:1, :] += jnp.sum(y, axis=0, keepdims=True)
    st_ref[1:2, :] += jnp.sum(y * y, axis=0, keepdims=True)


def _p4_body(y_ref, a_ref, c_ref, o_ref):
    h = jnp.maximum(y_ref[...] * a_ref[...] + c_ref[...], 0.0)
    o_ref[...] = jnp.max(h.reshape(GB, K, C2), axis=1)


def _run_p1(x, cent, w0t):
    return pl.pallas_call(
        _p1_body,
        grid=(RTOT // RB,),
        in_specs=[
            pl.BlockSpec((RB, CPAD), lambda i: (i, 0)),
            pl.BlockSpec((GB, CPAD), lambda i: (i, 0)),
            pl.BlockSpec((CPAD, CPAD), lambda i: (0, 0)),
        ],
        out_specs=[
            pl.BlockSpec((RB, CPAD), lambda i: (i, 0)),
            pl.BlockSpec((8, CPAD), lambda i: (0, 0)),
        ],
        out_shape=[
            jax.ShapeDtypeStruct((RTOT, CPAD), jnp.float32),
            jax.ShapeDtypeStruct((8, CPAD), jnp.float32),
        ],
    )(x, cent, w0t)


def _run_mid(y, a, c, wt, cout):
    cin = y.shape[1]
    return pl.pallas_call(
        _mid_body,
        grid=(RTOT // RB,),
        in_specs=[
            pl.BlockSpec((RB, cin), lambda i: (i, 0)),
            pl.BlockSpec((1, cin), lambda i: (0, 0)),
            pl.BlockSpec((1, cin), lambda i: (0, 0)),
            pl.BlockSpec((cin, cout), lambda i: (0, 0)),
        ],
        out_specs=[
            pl.BlockSpec((RB, cout), lambda i: (i, 0)),
            pl.BlockSpec((8, cout), lambda i: (0, 0)),
        ],
        out_shape=[
            jax.ShapeDtypeStruct((RTOT, cout), jnp.float32),
            jax.ShapeDtypeStruct((8, cout), jnp.float32),
        ],
    )(y, a, c, wt)


def _run_p4(y, a, c):
    return pl.pallas_call(
        _p4_body,
        grid=(RTOT // RB,),
        in_specs=[
            pl.BlockSpec((RB, C2), lambda i: (i, 0)),
            pl.BlockSpec((1, C2), lambda i: (0, 0)),
            pl.BlockSpec((1, C2), lambda i: (0, 0)),
        ],
        out_specs=pl.BlockSpec((GB, C2), lambda i: (i, 0)),
        out_shape=jax.ShapeDtypeStruct((B * S, C2), jnp.float32),
    )(y, a, c)


def _affine(st, gamma, beta):
    mean = st[0] / RTOT
    var = st[1] / RTOT - mean * mean
    a = gamma / jnp.sqrt(var + EPS)
    return a[None, :], (beta - mean * (gamma / jnp.sqrt(var + EPS)))[None, :]


def kernel(xyz, points, W0, b0, gamma0, beta0, W1, b1, gamma1, beta1,
           W2, b2, gamma2, beta2):
    f32 = jnp.float32
    xyz_t = jnp.transpose(xyz, (0, 2, 1))
    pts_t = jnp.transpose(points, (0, 2, 1))
    feat = jnp.concatenate(
        [xyz_t, pts_t, jnp.zeros((B, N, CPAD - 19), f32)], axis=-1)
    feat = feat.reshape(B * N, CPAD)

    gathered = _sc_ball_gather(xyz, feat)

    new_xyz_t = xyz_t[:, :S, :]
    cent = jnp.concatenate(
        [new_xyz_t, jnp.zeros((B, S, CPAD - 3), f32)], axis=-1)
    cent = cent.reshape(B * S, CPAD)

    w0t = jnp.pad(W0, ((0, 0), (0, CPAD - 19))).T
    y0, st0 = _run_p1(gathered, cent, w0t)
    a0, c0 = _affine(st0[:, :32], gamma0, beta0)
    y1, st1 = _run_mid(y0, a0, c0, W1.T, 32)
    a1, c1 = _affine(st1[:, :32], gamma1, beta1)
    y2, st2 = _run_mid(y1, a1, c1, W2.T, C2)
    a2, c2 = _affine(st2[:, :C2], gamma2, beta2)
    out = _run_p4(y2, a2, c2)

    new_points = jnp.transpose(out.reshape(B, S, C2), (0, 2, 1))
    new_xyz_out = xyz[:, :, :S]
    return (new_xyz_out, new_points)


# speedup vs baseline: 1.1508x; 1.1508x over previous
import functools

import jax
import jax.numpy as jnp
import numpy as np
from jax import lax
from jax.experimental import pallas as pl
from jax.experimental.pallas import tpu as pltpu
from jax.experimental.pallas import tpu_sc as plsc

B = 4
N = 8192
S = 1024
K = 32
CPAD = 32
C2 = 64
R2 = np.float32(0.04)
EPS = 1e-5
NC, NS = 2, 16
NW = NC * NS
RPW = (B * S) // NW
WPB = NW // B
GC = 16
NCHUNK = N // 16
RTOT = B * S * K
UNROLL = 8
SCANP = 192
RB = 2048
GB = RB // K


def _bf16_round(v):
    u = plsc.bitcast(v, jnp.uint32)
    lsb = lax.shift_right_logical(u, jnp.uint32(16)) & jnp.uint32(1)
    r = (u + jnp.uint32(0x7FFF) + lsb) & jnp.uint32(0xFFFF0000)
    return plsc.bitcast(r, jnp.float32)


def _sc_body(xyz_hbm, feat_hbm, out_hbm, xyz_v, pb_v, b2_v, scan_v, idx_v,
             gbuf_v, sem):
    wid = lax.axis_index("s") * NC + lax.axis_index("c")
    b = wid // WPB
    s_base = (wid % WPB) * RPW
    pltpu.sync_copy(xyz_hbm.at[b], xyz_v)

    lane = jnp.arange(16, dtype=jnp.int32)

    def pre_body(ch, carry):
        n0 = ch * 16
        px = xyz_v[pl.ds(n0, 16)]
        py = xyz_v[pl.ds(N + n0, 16)]
        pz = xyz_v[pl.ds(2 * N + n0, 16)]
        pb_v[pl.ds(n0, 16)] = _bf16_round(px)
        pb_v[pl.ds(N + n0, 16)] = _bf16_round(py)
        pb_v[pl.ds(2 * N + n0, 16)] = _bf16_round(pz)
        b2_v[pl.ds(n0, 16)] = (px * px + py * py) + pz * pz
        return carry

    lax.fori_loop(0, NCHUNK, pre_body, 0)

    def centroid_body(ci, carry):
        s = s_base + ci
        sv = jnp.full((16,), s, jnp.int32)
        cx = plsc.load_gather(xyz_v, [sv])
        cy = plsc.load_gather(xyz_v, [sv + N])
        cz = plsc.load_gather(xyz_v, [sv + 2 * N])
        a2 = (cx * cx + cy * cy) + cz * cz
        cbx = _bf16_round(cx)
        cby = _bf16_round(cy)
        cbz = _bf16_round(cz)
        c2x = -2.0 * cbx
        c2y = -2.0 * cby
        c2z = -2.0 * cbz
        rowbase = ci * K
        crowbase = ci * SCANP

        def cond(c):
            chunk, count = c
            return jnp.logical_and(count < K, chunk < NCHUNK // UNROLL)

        def body(c):
            chunk, count = c
            cnt = count
            for u in range(UNROLL):
                n0 = chunk * (16 * UNROLL) + u * 16
                q = (c2x * pb_v[pl.ds(n0, 16)] + c2y * pb_v[pl.ds(N + n0, 16)]
                     ) + c2z * pb_v[pl.ds(2 * N + n0, 16)]
                d2 = (q + a2) + b2_v[pl.ds(n0, 16)]
                m = d2 <= R2
                plsc.store_compressed(scan_v.at[pl.ds(crowbase + cnt, 16)],
                                      b * N + n0 + lane, mask=m)
                pcnt = plsc.all_reduce_population_count(m)
                cnt = cnt + pcnt[0]
            return chunk + 1, cnt

        _, cnt = lax.while_loop(cond, body, (jnp.int32(0), jnp.int32(0)))
        first = plsc.load_gather(scan_v, [jnp.full((16,), crowbase, jnp.int32)])
        for h in range(2):
            lv = lane + (h * 16)
            vals = scan_v[pl.ds(crowbase + h * 16, 16)]
            idx_v[pl.ds(rowbase + h * 16, 16)] = jnp.where(lv < cnt, vals, first)
        return carry

    lax.fori_loop(0, RPW, centroid_body, 0)

    out_base = (b * S + s_base) * K
    nrows = GC * K

    def gather_body(gi, carry):
        idx_sub = idx_v.at[pl.ds(gi * nrows, nrows)]
        pltpu.async_copy(feat_hbm.at[idx_sub], gbuf_v, sem).wait()
        pltpu.sync_copy(gbuf_v, out_hbm.at[pl.ds(out_base + gi * nrows, nrows)])
        return carry

    lax.fori_loop(0, RPW // GC, gather_body, 0)


def _sc_ball_gather(xyz, feat):
    mesh = plsc.VectorSubcoreMesh(
        core_axis_name="c", subcore_axis_name="s", num_cores=NC, num_subcores=NS)
    return pl.kernel(
        _sc_body,
        out_type=jax.ShapeDtypeStruct((RTOT, CPAD), jnp.float32),
        mesh=mesh,
        scratch_types=[
            pltpu.VMEM((3 * N,), jnp.float32),
            pltpu.VMEM((3 * N,), jnp.float32),
            pltpu.VMEM((N,), jnp.float32),
            pltpu.VMEM((RPW * SCANP,), jnp.int32),
            pltpu.VMEM((RPW * K,), jnp.int32),
            pltpu.VMEM((GC * K, CPAD), jnp.float32),
            pltpu.SemaphoreType.DMA,
        ],
        compiler_params=pltpu.CompilerParams(
            needs_layout_passes=False, use_tc_tiling_on_sc=False),
    )(xyz.reshape(B, 3 * N), feat)


def _p1_body(x_ref, cent_ref, w_ref, y_ref, st_ref):
    i = pl.program_id(0)
    x = x_ref[...]
    xc = x.reshape(GB, K, CPAD) - cent_ref[...][:, None, :]
    y = jnp.dot(xc.reshape(RB, CPAD), w_ref[...],
                preferred_element_type=jnp.float32)
    y_ref[...] = y

    @pl.when(i == 0)
    def _():
        st_ref[...] = jnp.zeros_like(st_ref)

    st_ref[0:1, :] += jnp.sum(y, axis=0, keepdims=True)
    st_ref[1:2, :] += jnp.sum(y * y, axis=0, keepdims=True)


def _mid_body(y_ref, a_ref, c_ref, w_ref, o_ref, st_ref):
    i = pl.program_id(0)
    h = jnp.maximum(y_ref[...] * a_ref[...] + c_ref[...], 0.0)
    y = jnp.dot(h, w_ref[...], preferred_element_type=jnp.float32)
    o_ref[...] = y

    @pl.when(i == 0)
    def _():
        st_ref[...] = jnp.zeros_like(st_ref)

    st_ref[0:1, :] += jnp.sum(y, axis=0, keepdims=True)
    st_ref[1:2, :] += jnp.sum(y * y, axis=0, keepdims=True)


def _p4_body(y_ref, a_ref, c_ref, o_ref):
    h = jnp.maximum(y_ref[...] * a_ref[...] + c_ref[...], 0.0)
    o_ref[...] = jnp.max(h.reshape(GB, K, C2), axis=1)


def _run_p1(x, cent, w0t):
    return pl.pallas_call(
        _p1_body,
        grid=(RTOT // RB,),
        in_specs=[
            pl.BlockSpec((RB, CPAD), lambda i: (i, 0)),
            pl.BlockSpec((GB, CPAD), lambda i: (i, 0)),
            pl.BlockSpec((CPAD, CPAD), lambda i: (0, 0)),
        ],
        out_specs=[
            pl.BlockSpec((RB, CPAD), lambda i: (i, 0)),
            pl.BlockSpec((8, CPAD), lambda i: (0, 0)),
        ],
        out_shape=[
            jax.ShapeDtypeStruct((RTOT, CPAD), jnp.float32),
            jax.ShapeDtypeStruct((8, CPAD), jnp.float32),
        ],
    )(x, cent, w0t)


def _run_mid(y, a, c, wt, cout):
    cin = y.shape[1]
    return pl.pallas_call(
        _mid_body,
        grid=(RTOT // RB,),
        in_specs=[
            pl.BlockSpec((RB, cin), lambda i: (i, 0)),
            pl.BlockSpec((1, cin), lambda i: (0, 0)),
            pl.BlockSpec((1, cin), lambda i: (0, 0)),
            pl.BlockSpec((cin, cout), lambda i: (0, 0)),
        ],
        out_specs=[
            pl.BlockSpec((RB, cout), lambda i: (i, 0)),
            pl.BlockSpec((8, cout), lambda i: (0, 0)),
        ],
        out_shape=[
            jax.ShapeDtypeStruct((RTOT, cout), jnp.float32),
            jax.ShapeDtypeStruct((8, cout), jnp.float32),
        ],
    )(y, a, c, wt)


def _run_p4(y, a, c):
    return pl.pallas_call(
        _p4_body,
        grid=(RTOT // RB,),
        in_specs=[
            pl.BlockSpec((RB, C2), lambda i: (i, 0)),
            pl.BlockSpec((1, C2), lambda i: (0, 0)),
            pl.BlockSpec((1, C2), lambda i: (0, 0)),
        ],
        out_specs=pl.BlockSpec((GB, C2), lambda i: (i, 0)),
        out_shape=jax.ShapeDtypeStruct((B * S, C2), jnp.float32),
    )(y, a, c)


def _affine(st, gamma, beta):
    mean = st[0] / RTOT
    var = st[1] / RTOT - mean * mean
    a = gamma / jnp.sqrt(var + EPS)
    return a[None, :], (beta - mean * (gamma / jnp.sqrt(var + EPS)))[None, :]


def kernel(xyz, points, W0, b0, gamma0, beta0, W1, b1, gamma1, beta1,
           W2, b2, gamma2, beta2):
    f32 = jnp.float32
    xyz_t = jnp.transpose(xyz, (0, 2, 1))
    pts_t = jnp.transpose(points, (0, 2, 1))
    feat = jnp.concatenate(
        [xyz_t, pts_t, jnp.zeros((B, N, CPAD - 19), f32)], axis=-1)
    feat = feat.reshape(B * N, CPAD)

    gathered = _sc_ball_gather(xyz, feat)

    new_xyz_t = xyz_t[:, :S, :]
    cent = jnp.concatenate(
        [new_xyz_t, jnp.zeros((B, S, CPAD - 3), f32)], axis=-1)
    cent = cent.reshape(B * S, CPAD)

    w0t = jnp.pad(W0, ((0, 0), (0, CPAD - 19))).T
    y0, st0 = _run_p1(gathered, cent, w0t)
    a0, c0 = _affine(st0[:, :32], gamma0, beta0)
    y1, st1 = _run_mid(y0, a0, c0, W1.T, 32)
    a1, c1 = _affine(st1[:, :32], gamma1, beta1)
    y2, st2 = _run_mid(y1, a1, c1, W2.T, C2)
    a2, c2 = _affine(st2[:, :C2], gamma2, beta2)
    out = _run_p4(y2, a2, c2)

    new_points = jnp.transpose(out.reshape(B, S, C2), (0, 2, 1))
    new_xyz_out = xyz[:, :, :S]
    return (new_xyz_out, new_points)


# speedup vs baseline: 1.4379x; 1.2494x over previous
import functools

import jax
import jax.numpy as jnp
import numpy as np
from jax import lax
from jax.experimental import pallas as pl
from jax.experimental.pallas import tpu as pltpu
from jax.experimental.pallas import tpu_sc as plsc

B = 4
N = 8192
S = 1024
K = 32
CPAD = 32
C2 = 64
R2 = np.float32(0.04)
EPS = 1e-5
NC, NS = 2, 16
NW = NC * NS
RPW = (B * S) // NW
WPB = NW // B
GC = 16
NCHUNK = N // 16
RTOT = B * S * K
UNROLL = 8
SCANP = 192
RB = 2048
GB = RB // K


def _bf16_round(v):
    u = plsc.bitcast(v, jnp.uint32)
    lsb = lax.shift_right_logical(u, jnp.uint32(16)) & jnp.uint32(1)
    r = (u + jnp.uint32(0x7FFF) + lsb) & jnp.uint32(0xFFFF0000)
    return plsc.bitcast(r, jnp.float32)


def _sc_body(xyz_hbm, feat_hbm, out_hbm, xyz_v, pb_v, b2_v, scan_v, idx_v,
             gbuf_v, sem):
    wid = lax.axis_index("s") * NC + lax.axis_index("c")
    b = wid // WPB
    s_base = (wid % WPB) * RPW
    pltpu.sync_copy(xyz_hbm.at[b], xyz_v)

    lane = jnp.arange(16, dtype=jnp.int32)

    def pre_body(ch, carry):
        n0 = ch * 16
        px = xyz_v[pl.ds(n0, 16)]
        py = xyz_v[pl.ds(N + n0, 16)]
        pz = xyz_v[pl.ds(2 * N + n0, 16)]
        pb_v[pl.ds(n0, 16)] = _bf16_round(px)
        pb_v[pl.ds(N + n0, 16)] = _bf16_round(py)
        pb_v[pl.ds(2 * N + n0, 16)] = _bf16_round(pz)
        b2_v[pl.ds(n0, 16)] = (px * px + py * py) + pz * pz
        return carry

    lax.fori_loop(0, NCHUNK, pre_body, 0)

    def centroid_body(ci, carry):
        s = s_base + ci
        sv = jnp.full((16,), s, jnp.int32)
        cx = plsc.load_gather(xyz_v, [sv])
        cy = plsc.load_gather(xyz_v, [sv + N])
        cz = plsc.load_gather(xyz_v, [sv + 2 * N])
        a2 = (cx * cx + cy * cy) + cz * cz
        cbx = _bf16_round(cx)
        cby = _bf16_round(cy)
        cbz = _bf16_round(cz)
        c2x = -2.0 * cbx
        c2y = -2.0 * cby
        c2z = -2.0 * cbz
        rowbase = ci * K
        crowbase = ci * SCANP

        def cond(c):
            chunk, count = c
            return jnp.logical_and(count < K, chunk < NCHUNK // UNROLL)

        def body(c):
            chunk, count = c
            masks, pcs = [], []
            for u in range(UNROLL):
                n0 = chunk * (16 * UNROLL) + u * 16
                q = (c2x * pb_v[pl.ds(n0, 16)] + c2y * pb_v[pl.ds(N + n0, 16)]
                     ) + c2z * pb_v[pl.ds(2 * N + n0, 16)]
                d2 = (q + a2) + b2_v[pl.ds(n0, 16)]
                m = d2 <= R2
                masks.append(m)
                pcs.append(plsc.all_reduce_population_count(m)[0])
            cnt = count
            for u in range(UNROLL):
                n0 = chunk * (16 * UNROLL) + u * 16
                plsc.store_compressed(scan_v.at[pl.ds(crowbase + cnt, 16)],
                                      b * N + n0 + lane, mask=masks[u])
                cnt = cnt + pcs[u]
            return chunk + 1, cnt

        _, cnt = lax.while_loop(cond, body, (jnp.int32(0), jnp.int32(0)))
        first = plsc.load_gather(scan_v, [jnp.full((16,), crowbase, jnp.int32)])
        for h in range(2):
            lv = lane + (h * 16)
            vals = scan_v[pl.ds(crowbase + h * 16, 16)]
            idx_v[pl.ds(rowbase + h * 16, 16)] = jnp.where(lv < cnt, vals, first)
        return carry

    lax.fori_loop(0, RPW, centroid_body, 0)

    out_base = (b * S + s_base) * K
    nrows = GC * K

    def gather_body(gi, carry):
        idx_sub = idx_v.at[pl.ds(gi * nrows, nrows)]
        pltpu.async_copy(feat_hbm.at[idx_sub], gbuf_v, sem).wait()
        pltpu.sync_copy(gbuf_v, out_hbm.at[pl.ds(out_base + gi * nrows, nrows)])
        return carry

    lax.fori_loop(0, RPW // GC, gather_body, 0)


def _sc_ball_gather(xyz, feat):
    mesh = plsc.VectorSubcoreMesh(
        core_axis_name="c", subcore_axis_name="s", num_cores=NC, num_subcores=NS)
    return pl.kernel(
        _sc_body,
        out_type=jax.ShapeDtypeStruct((RTOT, CPAD), jnp.float32),
        mesh=mesh,
        scratch_types=[
            pltpu.VMEM((3 * N,), jnp.float32),
            pltpu.VMEM((3 * N,), jnp.float32),
            pltpu.VMEM((N,), jnp.float32),
            pltpu.VMEM((RPW * SCANP,), jnp.int32),
            pltpu.VMEM((RPW * K,), jnp.int32),
            pltpu.VMEM((GC * K, CPAD), jnp.float32),
            pltpu.SemaphoreType.DMA,
        ],
        compiler_params=pltpu.CompilerParams(
            needs_layout_passes=False, use_tc_tiling_on_sc=False),
    )(xyz.reshape(B, 3 * N), feat)


def _p1_body(x_ref, cent_ref, w_ref, y_ref, st_ref):
    i = pl.program_id(0)
    x = x_ref[...]
    xc = x.reshape(GB, K, CPAD) - cent_ref[...][:, None, :]
    y = jnp.dot(xc.reshape(RB, CPAD), w_ref[...],
                preferred_element_type=jnp.float32)
    y_ref[...] = y

    @pl.when(i == 0)
    def _():
        st_ref[...] = jnp.zeros_like(st_ref)

    st_ref[0:1, :] += jnp.sum(y, axis=0, keepdims=True)
    st_ref[1:2, :] += jnp.sum(y * y, axis=0, keepdims=True)


def _mid_body(y_ref, a_ref, c_ref, w_ref, o_ref, st_ref):
    i = pl.program_id(0)
    h = jnp.maximum(y_ref[...] * a_ref[...] + c_ref[...], 0.0)
    y = jnp.dot(h, w_ref[...], preferred_element_type=jnp.float32)
    o_ref[...] = y

    @pl.when(i == 0)
    def _():
        st_ref[...] = jnp.zeros_like(st_ref)

    st_ref[0:1, :] += jnp.sum(y, axis=0, keepdims=True)
    st_ref[1:2, :] += jnp.sum(y * y, axis=0, keepdims=True)


def _p4_body(y_ref, a_ref, c_ref, o_ref):
    h = jnp.maximum(y_ref[...] * a_ref[...] + c_ref[...], 0.0)
    o_ref[...] = jnp.max(h.reshape(GB, K, C2), axis=1)


def _run_p1(x, cent, w0t):
    return pl.pallas_call(
        _p1_body,
        grid=(RTOT // RB,),
        in_specs=[
            pl.BlockSpec((RB, CPAD), lambda i: (i, 0)),
            pl.BlockSpec((GB, CPAD), lambda i: (i, 0)),
            pl.BlockSpec((CPAD, CPAD), lambda i: (0, 0)),
        ],
        out_specs=[
            pl.BlockSpec((RB, CPAD), lambda i: (i, 0)),
            pl.BlockSpec((8, CPAD), lambda i: (0, 0)),
        ],
        out_shape=[
            jax.ShapeDtypeStruct((RTOT, CPAD), jnp.float32),
            jax.ShapeDtypeStruct((8, CPAD), jnp.float32),
        ],
    )(x, cent, w0t)


def _run_mid(y, a, c, wt, cout):
    cin = y.shape[1]
    return pl.pallas_call(
        _mid_body,
        grid=(RTOT // RB,),
        in_specs=[
            pl.BlockSpec((RB, cin), lambda i: (i, 0)),
            pl.BlockSpec((1, cin), lambda i: (0, 0)),
            pl.BlockSpec((1, cin), lambda i: (0, 0)),
            pl.BlockSpec((cin, cout), lambda i: (0, 0)),
        ],
        out_specs=[
            pl.BlockSpec((RB, cout), lambda i: (i, 0)),
            pl.BlockSpec((8, cout), lambda i: (0, 0)),
        ],
        out_shape=[
            jax.ShapeDtypeStruct((RTOT, cout), jnp.float32),
            jax.ShapeDtypeStruct((8, cout), jnp.float32),
        ],
    )(y, a, c, wt)


def _run_p4(y, a, c):
    return pl.pallas_call(
        _p4_body,
        grid=(RTOT // RB,),
        in_specs=[
            pl.BlockSpec((RB, C2), lambda i: (i, 0)),
            pl.BlockSpec((1, C2), lambda i: (0, 0)),
            pl.BlockSpec((1, C2), lambda i: (0, 0)),
        ],
        out_specs=pl.BlockSpec((GB, C2), lambda i: (i, 0)),
        out_shape=jax.ShapeDtypeStruct((B * S, C2), jnp.float32),
    )(y, a, c)


def _affine(st, gamma, beta):
    mean = st[0] / RTOT
    var = st[1] / RTOT - mean * mean
    a = gamma / jnp.sqrt(var + EPS)
    return a[None, :], (beta - mean * (gamma / jnp.sqrt(var + EPS)))[None, :]


def kernel(xyz, points, W0, b0, gamma0, beta0, W1, b1, gamma1, beta1,
           W2, b2, gamma2, beta2):
    f32 = jnp.float32
    xyz_t = jnp.transpose(xyz, (0, 2, 1))
    pts_t = jnp.transpose(points, (0, 2, 1))
    feat = jnp.concatenate(
        [xyz_t, pts_t, jnp.zeros((B, N, CPAD - 19), f32)], axis=-1)
    feat = feat.reshape(B * N, CPAD)

    gathered = _sc_ball_gather(xyz, feat)

    new_xyz_t = xyz_t[:, :S, :]
    cent = jnp.concatenate(
        [new_xyz_t, jnp.zeros((B, S, CPAD - 3), f32)], axis=-1)
    cent = cent.reshape(B * S, CPAD)

    w0t = jnp.pad(W0, ((0, 0), (0, CPAD - 19))).T
    y0, st0 = _run_p1(gathered, cent, w0t)
    a0, c0 = _affine(st0[:, :32], gamma0, beta0)
    y1, st1 = _run_mid(y0, a0, c0, W1.T, 32)
    a1, c1 = _affine(st1[:, :32], gamma1, beta1)
    y2, st2 = _run_mid(y1, a1, c1, W2.T, C2)
    a2, c2 = _affine(st2[:, :C2], gamma2, beta2)
    out = _run_p4(y2, a2, c2)

    new_points = jnp.transpose(out.reshape(B, S, C2), (0, 2, 1))
    new_xyz_out = xyz[:, :, :S]
    return (new_xyz_out, new_points)


# speedup vs baseline: 1.4836x; 1.0318x over previous
import functools

import jax
import jax.numpy as jnp
import numpy as np
from jax import lax
from jax.experimental import pallas as pl
from jax.experimental.pallas import tpu as pltpu
from jax.experimental.pallas import tpu_sc as plsc

B = 4
N = 8192
S = 1024
K = 32
CPAD = 32
C2 = 64
R2 = np.float32(0.04)
EPS = 1e-5
NC, NS = 2, 16
NW = NC * NS
RPW = (B * S) // NW
WPB = NW // B
GC = 16
NCHUNK = N // 16
RTOT = B * S * K
UNROLL = 16
SCANP = 320
RB = 2048
GB = RB // K


def _bf16_round(v):
    u = plsc.bitcast(v, jnp.uint32)
    lsb = lax.shift_right_logical(u, jnp.uint32(16)) & jnp.uint32(1)
    r = (u + jnp.uint32(0x7FFF) + lsb) & jnp.uint32(0xFFFF0000)
    return plsc.bitcast(r, jnp.float32)


def _sc_body(xyz_hbm, feat_hbm, out_hbm, xyz_v, pb_v, b2_v, scan_v, idx_v,
             gbuf_v, sem):
    wid = lax.axis_index("s") * NC + lax.axis_index("c")
    b = wid // WPB
    s_base = (wid % WPB) * RPW
    pltpu.sync_copy(xyz_hbm.at[b], xyz_v)

    lane = jnp.arange(16, dtype=jnp.int32)

    def pre_body(ch, carry):
        n0 = ch * 16
        px = xyz_v[pl.ds(n0, 16)]
        py = xyz_v[pl.ds(N + n0, 16)]
        pz = xyz_v[pl.ds(2 * N + n0, 16)]
        pb_v[pl.ds(n0, 16)] = _bf16_round(px)
        pb_v[pl.ds(N + n0, 16)] = _bf16_round(py)
        pb_v[pl.ds(2 * N + n0, 16)] = _bf16_round(pz)
        b2_v[pl.ds(n0, 16)] = (px * px + py * py) + pz * pz
        return carry

    lax.fori_loop(0, NCHUNK, pre_body, 0)

    def centroid_body(ci, carry):
        s = s_base + ci
        sv = jnp.full((16,), s, jnp.int32)
        cx = plsc.load_gather(xyz_v, [sv])
        cy = plsc.load_gather(xyz_v, [sv + N])
        cz = plsc.load_gather(xyz_v, [sv + 2 * N])
        a2 = (cx * cx + cy * cy) + cz * cz
        cbx = _bf16_round(cx)
        cby = _bf16_round(cy)
        cbz = _bf16_round(cz)
        c2x = -2.0 * cbx
        c2y = -2.0 * cby
        c2z = -2.0 * cbz
        rowbase = ci * K
        crowbase = ci * SCANP

        def cond(c):
            chunk, count = c
            return jnp.logical_and(count < K, chunk < NCHUNK // UNROLL)

        def body(c):
            chunk, count = c
            masks, pcs = [], []
            for u in range(UNROLL):
                n0 = chunk * (16 * UNROLL) + u * 16
                q = (c2x * pb_v[pl.ds(n0, 16)] + c2y * pb_v[pl.ds(N + n0, 16)]
                     ) + c2z * pb_v[pl.ds(2 * N + n0, 16)]
                d2 = (q + a2) + b2_v[pl.ds(n0, 16)]
                m = d2 <= R2
                masks.append(m)
                pcs.append(plsc.all_reduce_population_count(m)[0])
            cnt = count
            for u in range(UNROLL):
                n0 = chunk * (16 * UNROLL) + u * 16
                plsc.store_compressed(scan_v.at[pl.ds(crowbase + cnt, 16)],
                                      b * N + n0 + lane, mask=masks[u])
                cnt = cnt + pcs[u]
            return chunk + 1, cnt

        _, cnt = lax.while_loop(cond, body, (jnp.int32(0), jnp.int32(0)))
        first = plsc.load_gather(scan_v, [jnp.full((16,), crowbase, jnp.int32)])
        for h in range(2):
            lv = lane + (h * 16)
            vals = scan_v[pl.ds(crowbase + h * 16, 16)]
            idx_v[pl.ds(rowbase + h * 16, 16)] = jnp.where(lv < cnt, vals, first)
        return carry

    lax.fori_loop(0, RPW, centroid_body, 0)

    out_base = (b * S + s_base) * K
    nrows = GC * K

    def gather_body(gi, carry):
        idx_sub = idx_v.at[pl.ds(gi * nrows, nrows)]
        pltpu.async_copy(feat_hbm.at[idx_sub], gbuf_v, sem).wait()
        pltpu.sync_copy(gbuf_v, out_hbm.at[pl.ds(out_base + gi * nrows, nrows)])
        return carry

    lax.fori_loop(0, RPW // GC, gather_body, 0)


def _sc_ball_gather(xyz, feat):
    mesh = plsc.VectorSubcoreMesh(
        core_axis_name="c", subcore_axis_name="s", num_cores=NC, num_subcores=NS)
    return pl.kernel(
        _sc_body,
        out_type=jax.ShapeDtypeStruct((RTOT, CPAD), jnp.float32),
        mesh=mesh,
        scratch_types=[
            pltpu.VMEM((3 * N,), jnp.float32),
            pltpu.VMEM((3 * N,), jnp.float32),
            pltpu.VMEM((N,), jnp.float32),
            pltpu.VMEM((RPW * SCANP,), jnp.int32),
            pltpu.VMEM((RPW * K,), jnp.int32),
            pltpu.VMEM((GC * K, CPAD), jnp.float32),
            pltpu.SemaphoreType.DMA,
        ],
        compiler_params=pltpu.CompilerParams(
            needs_layout_passes=False, use_tc_tiling_on_sc=False),
    )(xyz.reshape(B, 3 * N), feat)


def _p1_body(x_ref, cent_ref, w_ref, y_ref, st_ref):
    i = pl.program_id(0)
    x = x_ref[...]
    xc = x.reshape(GB, K, CPAD) - cent_ref[...][:, None, :]
    y = jnp.dot(xc.reshape(RB, CPAD), w_ref[...],
                preferred_element_type=jnp.float32)
    y_ref[...] = y

    @pl.when(i == 0)
    def _():
        st_ref[...] = jnp.zeros_like(st_ref)

    st_ref[0:1, :] += jnp.sum(y, axis=0, keepdims=True)
    st_ref[1:2, :] += jnp.sum(y * y, axis=0, keepdims=True)


def _mid_body(y_ref, a_ref, c_ref, w_ref, o_ref, st_ref):
    i = pl.program_id(0)
    h = jnp.maximum(y_ref[...] * a_ref[...] + c_ref[...], 0.0)
    y = jnp.dot(h, w_ref[...], preferred_element_type=jnp.float32)
    o_ref[...] = y

    @pl.when(i == 0)
    def _():
        st_ref[...] = jnp.zeros_like(st_ref)

    st_ref[0:1, :] += jnp.sum(y, axis=0, keepdims=True)
    st_ref[1:2, :] += jnp.sum(y * y, axis=0, keepdims=True)


def _p4_body(y_ref, a_ref, c_ref, o_ref):
    h = jnp.maximum(y_ref[...] * a_ref[...] + c_ref[...], 0.0)
    o_ref[...] = jnp.max(h.reshape(GB, K, C2), axis=1)


def _run_p1(x, cent, w0t):
    return pl.pallas_call(
        _p1_body,
        grid=(RTOT // RB,),
        in_specs=[
            pl.BlockSpec((RB, CPAD), lambda i: (i, 0)),
            pl.BlockSpec((GB, CPAD), lambda i: (i, 0)),
            pl.BlockSpec((CPAD, CPAD), lambda i: (0, 0)),
        ],
        out_specs=[
            pl.BlockSpec((RB, CPAD), lambda i: (i, 0)),
            pl.BlockSpec((8, CPAD), lambda i: (0, 0)),
        ],
        out_shape=[
            jax.ShapeDtypeStruct((RTOT, CPAD), jnp.float32),
            jax.ShapeDtypeStruct((8, CPAD), jnp.float32),
        ],
    )(x, cent, w0t)


def _run_mid(y, a, c, wt, cout):
    cin = y.shape[1]
    return pl.pallas_call(
        _mid_body,
        grid=(RTOT // RB,),
        in_specs=[
            pl.BlockSpec((RB, cin), lambda i: (i, 0)),
            pl.BlockSpec((1, cin), lambda i: (0, 0)),
            pl.BlockSpec((1, cin), lambda i: (0, 0)),
            pl.BlockSpec((cin, cout), lambda i: (0, 0)),
        ],
        out_specs=[
            pl.BlockSpec((RB, cout), lambda i: (i, 0)),
            pl.BlockSpec((8, cout), lambda i: (0, 0)),
        ],
        out_shape=[
            jax.ShapeDtypeStruct((RTOT, cout), jnp.float32),
            jax.ShapeDtypeStruct((8, cout), jnp.float32),
        ],
    )(y, a, c, wt)


def _run_p4(y, a, c):
    return pl.pallas_call(
        _p4_body,
        grid=(RTOT // RB,),
        in_specs=[
            pl.BlockSpec((RB, C2), lambda i: (i, 0)),
            pl.BlockSpec((1, C2), lambda i: (0, 0)),
            pl.BlockSpec((1, C2), lambda i: (0, 0)),
        ],
        out_specs=pl.BlockSpec((GB, C2), lambda i: (i, 0)),
        out_shape=jax.ShapeDtypeStruct((B * S, C2), jnp.float32),
    )(y, a, c)


def _affine(st, gamma, beta):
    mean = st[0] / RTOT
    var = st[1] / RTOT - mean * mean
    a = gamma / jnp.sqrt(var + EPS)
    return a[None, :], (beta - mean * (gamma / jnp.sqrt(var + EPS)))[None, :]


def kernel(xyz, points, W0, b0, gamma0, beta0, W1, b1, gamma1, beta1,
           W2, b2, gamma2, beta2):
    f32 = jnp.float32
    xyz_t = jnp.transpose(xyz, (0, 2, 1))
    pts_t = jnp.transpose(points, (0, 2, 1))
    feat = jnp.concatenate(
        [xyz_t, pts_t, jnp.zeros((B, N, CPAD - 19), f32)], axis=-1)
    feat = feat.reshape(B * N, CPAD)

    gathered = _sc_ball_gather(xyz, feat)

    new_xyz_t = xyz_t[:, :S, :]
    cent = jnp.concatenate(
        [new_xyz_t, jnp.zeros((B, S, CPAD - 3), f32)], axis=-1)
    cent = cent.reshape(B * S, CPAD)

    w0t = jnp.pad(W0, ((0, 0), (0, CPAD - 19))).T
    y0, st0 = _run_p1(gathered, cent, w0t)
    a0, c0 = _affine(st0[:, :32], gamma0, beta0)
    y1, st1 = _run_mid(y0, a0, c0, W1.T, 32)
    a1, c1 = _affine(st1[:, :32], gamma1, beta1)
    y2, st2 = _run_mid(y1, a1, c1, W2.T, C2)
    a2, c2 = _affine(st2[:, :C2], gamma2, beta2)
    out = _run_p4(y2, a2, c2)

    new_points = jnp.transpose(out.reshape(B, S, C2), (0, 2, 1))
    new_xyz_out = xyz[:, :, :S]
    return (new_xyz_out, new_points)
